# baseline (speedup 1.0000x reference)
import functools

import jax
import jax.numpy as jnp
from jax import lax
from jax.experimental import pallas as pl
from jax.experimental.pallas import tpu as pltpu
from jax.experimental.pallas import tpu_sc as plsc

_INFO = plsc.get_sparse_core_info()
_NC = _INFO.num_cores
_NS = _INFO.num_subcores
_NW = _NC * _NS

_S = 50
_T = 16384
_D = 32
_TL = 128
_DS = 8
_NDT = _D // _DS
_TB = 512
_NQ = _T // _TB
_NBLK = _S * _NQ
_BPW = _NBLK // _NW

_DT_RUN = _DS * _TB
_TBUF = _D * _TB + 256

_V = 1_000_000
_VC = 512
_NCH = 1953
_CPW = 62
_CSTEP = 61
_CFL = _VC * _D
_OBUF = _CFL + 256


def _make_table_transpose():
    mesh = plsc.VectorSubcoreMesh(core_axis_name="c", subcore_axis_name="s")

    @functools.partial(
        pl.kernel,
        mesh=mesh,
        out_type=jax.ShapeDtypeStruct((_V * _D,), jnp.float32),
        scratch_types=[
            pltpu.VMEM((_D, _VC), jnp.float32),
            pltpu.VMEM((_D, _VC), jnp.float32),
            pltpu.VMEM((_OBUF,), jnp.float32),
            pltpu.VMEM((_OBUF,), jnp.float32),
            pltpu.SemaphoreType.DMA,
            pltpu.SemaphoreType.DMA,
            pltpu.SemaphoreType.DMA,
            pltpu.SemaphoreType.DMA,
        ],
        compiler_params=pltpu.CompilerParams(use_tc_tiling_on_sc=False,
                                             needs_layout_passes=False),
    )
    def tw(wt_hbm, out_hbm, c0, c1, o0, o1, g0, g1, s0, s1):
        wid = lax.axis_index("s") * _NC + lax.axis_index("c")
        start = wid * _CSTEP
        cb = (c0, c1)
        ob = (o0, o1)
        gs = (g0, g1)
        ss = (s0, s1)

        lanes = lax.iota(jnp.int32, 16)
        col_lo = [lax.rem(lanes + k, 16) for k in range(16)]
        col_hi = [c + 16 for c in col_lo]
        dst32 = [lanes * _D + c for c in col_lo]

        def wt_src(k):
            return wt_hbm.at[:, pl.ds((start + k) * _VC, _VC)]

        def out_dst(k):
            return out_hbm.at[pl.ds((start + k) * _CFL, _CFL)]

        def transpose_chunk(cbuf, obuf):
            def trv(vg, _):
                vv = vg * 16
                cv = lanes + vv
                for d0 in (0, 16):
                    oref = obuf.at[pl.ds(vv * _D + d0, 512)]
                    cols = col_lo if d0 == 0 else col_hi
                    for q in range(16):
                        plsc.store_scatter(
                            oref, [dst32[q]],
                            plsc.load_gather(cbuf, [cols[q], cv]))
                return 0

            lax.fori_loop(0, _VC // 16, trv, 0)

        pltpu.async_copy(wt_src(0), c0, g0)
        pltpu.async_copy(wt_src(1), c1, g1)

        def step(k2, _):
            for b in range(2):
                k = k2 * 2 + b
                cbuf, obuf, gsem, ssem = cb[b], ob[b], gs[b], ss[b]
                pltpu.make_async_copy(wt_src(k), cbuf, gsem).wait()

                @pl.when(k >= 2)
                def _():
                    pltpu.make_async_copy(obuf.at[pl.ds(0, _CFL)],
                                          out_dst(k - 2), ssem).wait()

                transpose_chunk(cbuf, obuf)
                pltpu.async_copy(obuf.at[pl.ds(0, _CFL)], out_dst(k), ssem)

                @pl.when(k + 2 < _CPW)
                def _():
                    pltpu.async_copy(wt_src(k + 2), cbuf, gsem)

            return 0

        lax.fori_loop(0, _CPW // 2, step, 0)
        for b in range(2):
            pltpu.make_async_copy(ob[b].at[pl.ds(0, _CFL)],
                                  out_dst(_CPW - 2 + b), ss[b]).wait()

        @pl.when(wid == _NW - 1)
        def _():
            pltpu.sync_copy(wt_hbm.at[:, pl.ds(_V - _VC, _VC)], c0)
            transpose_chunk(c0, o0)
            pltpu.sync_copy(o0.at[pl.ds(0, _CFL)],
                            out_hbm.at[pl.ds((_V - _VC) * _D, _CFL)])

    return tw


def _make_gather():
    mesh = plsc.VectorSubcoreMesh(core_axis_name="c", subcore_axis_name="s")

    @functools.partial(
        pl.kernel,
        mesh=mesh,
        out_type=jax.ShapeDtypeStruct((_S, _NDT, _T * _DS), jnp.float32),
        scratch_types=[
            pltpu.VMEM((_BPW, _TB), jnp.int32),
            pltpu.VMEM((_TB, _D), jnp.float32),
            pltpu.VMEM((_TB, _D), jnp.float32),
            pltpu.VMEM((_TBUF,), jnp.float32),
            pltpu.VMEM((_TBUF,), jnp.float32),
            pltpu.SemaphoreType.DMA,
            pltpu.SemaphoreType.DMA,
            pltpu.SemaphoreType.DMA,
            pltpu.SemaphoreType.DMA,
        ],
        compiler_params=pltpu.CompilerParams(use_tc_tiling_on_sc=False,
                                             needs_layout_passes=False),
    )
    def emb(table_hbm, idx_hbm, out_hbm, idx_v, rows0, rows1, tb0, tb1,
            gsem0, gsem1, ssem0, ssem1):
        wid = lax.axis_index("s") * _NC + lax.axis_index("c")
        base = wid * _BPW
        pltpu.sync_copy(idx_hbm.at[wid], idx_v)

        rows = (rows0, rows1)
        tbs = (tb0, tb1)
        gsems = (gsem0, gsem1)
        ssems = (ssem0, ssem1)

        lanes = lax.iota(jnp.int32, 16)
        col_k = [lax.rem(lanes + k, 16) for k in range(16)]
        col_hi_k = [c + 16 for c in col_k]
        dst_k = [(c // 8) * 4096 + lax.rem(c, 8) * _TL + lanes for c in col_k]
        _HI = 8192
        _SPAN = 5120

        def start_gather(k, rbuf, gsem):
            for h in range(2):
                pltpu.async_copy(
                    table_hbm.at[idx_v.at[k, pl.ds(h * 256, 256)]],
                    rbuf.at[pl.ds(h * 256, 256)], gsem)

        def wait_gather(k, rbuf, gsem):
            for h in range(2):
                pltpu.make_async_copy(
                    table_hbm.at[idx_v.at[k, pl.ds(h * 256, 256)]],
                    rbuf.at[pl.ds(h * 256, 256)], gsem).wait()

        start_gather(0, rows0, gsem0)
        start_gather(1, rows1, gsem1)

        def stores(bid, tbuf, ssem, wait):
            s = bid // _NQ
            q = lax.rem(bid, _NQ)
            for dt in range(_NDT):
                src = tbuf.at[pl.ds(dt * _DT_RUN, _DT_RUN)]
                dst = out_hbm.at[s, dt, pl.ds(q * _DT_RUN, _DT_RUN)]
                if wait:
                    pltpu.make_async_copy(src, dst, ssem).wait()
                else:
                    pltpu.async_copy(src, dst, ssem)

        def step(k2, _):
            for b in range(2):
                k = k2 * 2 + b
                rbuf, tbuf, gsem, ssem = rows[b], tbs[b], gsems[b], ssems[b]
                wait_gather(k, rbuf, gsem)

                @pl.when(k >= 2)
                def _():
                    stores(base + k - 2, tbuf, ssem, wait=True)

                def tr(tg, _):
                    t0 = tg * 16
                    off = (tg // 8) * 1024 + lax.rem(tg, 8) * 16
                    rv = lanes + t0
                    lo_ref = tbuf.at[pl.ds(off, _SPAN)]
                    hi_ref = tbuf.at[pl.ds(off + _HI, _SPAN)]
                    for k in range(16):
                        plsc.store_scatter(
                            lo_ref, [dst_k[k]],
                            plsc.load_gather(rbuf, [rv, col_k[k]]))
                        plsc.store_scatter(
                            hi_ref, [dst_k[k]],
                            plsc.load_gather(rbuf, [rv, col_hi_k[k]]))
                    return 0

                lax.fori_loop(0, _TB // 16, tr, 0)

                stores(base + k, tbuf, ssem, wait=False)

                @pl.when(k + 2 < _BPW)
                def _():
                    start_gather(k + 2, rbuf, gsem)

            return 0

        lax.fori_loop(0, _BPW // 2, step, 0)

        for b in range(2):
            stores(base + _BPW - 2 + b, tbs[b], ssems[b], wait=True)

    return emb


def kernel(token_ids, weight):
    trm = _make_table_transpose()(weight.T)
    table = trm.reshape(_V, _D)
    idsw = token_ids.astype(jnp.int32).T.reshape(_NW, _BPW, _TB)
    out3 = _make_gather()(table, idsw)
    out5 = out3.reshape(_S, _NDT, _T // _TL, _DS, _TL)
    return out5.transpose(2, 4, 0, 1, 3).reshape(_T, _S, _D)


# speedup vs baseline: 4.2056x; 4.2056x over previous
import functools

import jax
import jax.numpy as jnp
from jax import lax
from jax.experimental import pallas as pl
from jax.experimental.pallas import tpu as pltpu
from jax.experimental.pallas import tpu_sc as plsc

_INFO = plsc.get_sparse_core_info()
_NC = _INFO.num_cores
_NS = _INFO.num_subcores
_NW = _NC * _NS

_S = 50
_T = 16384
_D = 32
_TL = 128
_DS = 8
_NDT = _D // _DS
_TB = 512
_NQ = _T // _TB
_NBLK = _S * _NQ
_BPW = _NBLK // _NW

_DT_RUN = _DS * _TB
_TBUF = _D * _TB + 256


def _make_gather():
    mesh = plsc.VectorSubcoreMesh(core_axis_name="c", subcore_axis_name="s")

    @functools.partial(
        pl.kernel,
        mesh=mesh,
        out_type=jax.ShapeDtypeStruct((_S, _NDT, _T * _DS), jnp.float32),
        scratch_types=[
            pltpu.VMEM((_BPW, _TB), jnp.int32),
            pltpu.VMEM((_TB, _D), jnp.float32),
            pltpu.VMEM((_TB, _D), jnp.float32),
            pltpu.VMEM((_TBUF,), jnp.float32),
            pltpu.VMEM((_TBUF,), jnp.float32),
            pltpu.SemaphoreType.DMA,
            pltpu.SemaphoreType.DMA,
            pltpu.SemaphoreType.DMA,
            pltpu.SemaphoreType.DMA,
        ],
        compiler_params=pltpu.CompilerParams(use_tc_tiling_on_sc=False,
                                             needs_layout_passes=False),
    )
    def emb(table_hbm, idx_hbm, out_hbm, idx_v, rows0, rows1, tb0, tb1,
            gsem0, gsem1, ssem0, ssem1):
        wid = lax.axis_index("s") * _NC + lax.axis_index("c")
        base = wid * _BPW
        pltpu.sync_copy(idx_hbm.at[wid], idx_v)

        rows = (rows0, rows1)
        tbs = (tb0, tb1)
        gsems = (gsem0, gsem1)
        ssems = (ssem0, ssem1)

        lanes = lax.iota(jnp.int32, 16)
        col_k = [lax.rem(lanes + k, 16) for k in range(16)]
        col_hi_k = [c + 16 for c in col_k]
        dst_k = [(c // 8) * 4096 + lax.rem(c, 8) * _TL + lanes for c in col_k]
        _HI = 8192
        _SPAN = 5120

        def start_gather(k, rbuf, gsem):
            pltpu.async_copy(table_hbm.at[idx_v.at[k]], rbuf, gsem)

        def wait_gather(k, rbuf, gsem):
            pltpu.make_async_copy(table_hbm.at[idx_v.at[k]], rbuf,
                                  gsem).wait()

        start_gather(0, rows0, gsem0)
        start_gather(1, rows1, gsem1)

        def stores(bid, tbuf, ssem):
            s = bid // _NQ
            q = lax.rem(bid, _NQ)
            for dt in range(_NDT):
                pltpu.async_copy(
                    tbuf.at[pl.ds(dt * _DT_RUN, _DT_RUN)],
                    out_hbm.at[s, dt, pl.ds(q * _DT_RUN, _DT_RUN)], ssem)

        def drain_stores(tbuf, ssem):
            pltpu.make_async_copy(tbuf.at[pl.ds(0, _D * _TB)],
                                  out_hbm.at[0, 0, pl.ds(0, _D * _TB)],
                                  ssem).wait()

        def step(k2, _):
            for b in range(2):
                k = k2 * 2 + b
                rbuf, tbuf, gsem, ssem = rows[b], tbs[b], gsems[b], ssems[b]
                wait_gather(k, rbuf, gsem)

                @pl.when(k >= 2)
                def _():
                    drain_stores(tbuf, ssem)

                def tr(tg, _):
                    t0 = tg * 16
                    off = (tg // 8) * 1024 + lax.rem(tg, 8) * 16
                    rv = lanes + t0
                    lo_ref = tbuf.at[pl.ds(off, _SPAN)]
                    hi_ref = tbuf.at[pl.ds(off + _HI, _SPAN)]
                    for k in range(16):
                        plsc.store_scatter(
                            lo_ref, [dst_k[k]],
                            plsc.load_gather(rbuf, [rv, col_k[k]]))
                        plsc.store_scatter(
                            hi_ref, [dst_k[k]],
                            plsc.load_gather(rbuf, [rv, col_hi_k[k]]))
                    return 0

                lax.fori_loop(0, _TB // 16, tr, 0)

                stores(base + k, tbuf, ssem)

                @pl.when(k + 2 < _BPW)
                def _():
                    start_gather(k + 2, rbuf, gsem)

            return 0

        lax.fori_loop(0, _BPW // 2, step, 0)

        for b in range(2):
            drain_stores(tbs[b], ssems[b])

    return emb


def kernel(token_ids, weight):
    idsw = token_ids.astype(jnp.int32).T.reshape(_NW, _BPW, _TB)
    out3 = _make_gather()(weight, idsw)
    out5 = out3.reshape(_S, _NDT, _T // _TL, _DS, _TL)
    return out5.transpose(2, 4, 0, 1, 3).reshape(_T, _S, _D)
